# Initial kernel scaffold; baseline (speedup 1.0000x reference)
import functools
import math

import jax
import jax.numpy as jnp
from jax import lax
from jax.experimental import pallas as pl
from jax.experimental.pallas import tpu as pltpu
from jax.experimental.pallas import tpu_sc as plsc

_VOCAB = 100000
_D = 64
_SCALE = math.sqrt(_D)

_NC = 2
_NS = 16
_NW = _NC * _NS

_B = 4096 * 200
_CB = 128
_ROWS_PER_W = _B // _NW
_NCHUNK = _ROWS_PER_W // _CB
_NBUF = 4
_NOUTER = _NCHUNK // _NBUF


def _scale_body(w_ref, o_ref):
    o_ref[...] = w_ref[...] * _SCALE


def _scale_table(weight):
    return pl.pallas_call(
        _scale_body,
        out_shape=jax.ShapeDtypeStruct((_VOCAB, _D), jnp.float32),
        grid=(25,),
        in_specs=[pl.BlockSpec((_VOCAB // 25, _D), lambda i: (i, 0))],
        out_specs=pl.BlockSpec((_VOCAB // 25, _D), lambda i: (i, 0)),
    )(weight)


def _gather_body(idx_hbm, table_hbm, out_hbm, idx_v, rows_v, gsem, wsem):
    wid = lax.axis_index("s") * _NC + lax.axis_index("c")
    base_row = wid * _ROWS_PER_W

    pltpu.sync_copy(idx_hbm.at[pl.ds(wid * _NCHUNK, _NCHUNK)], idx_v)

    def _issue_gather(j, b):
        pltpu.async_copy(table_hbm.at[idx_v.at[j]], rows_v.at[b], gsem)

    def _wait_gather(j, b):
        pltpu.make_async_copy(table_hbm.at[idx_v.at[j]], rows_v.at[b], gsem).wait()

    def _issue_write(j, b):
        pltpu.async_copy(rows_v.at[b], out_hbm.at[pl.ds(base_row + j * _CB, _CB)], wsem)

    def _wait_write(j, b):
        pltpu.make_async_copy(rows_v.at[b], out_hbm.at[pl.ds(base_row + j * _CB, _CB)], wsem).wait()

    for b in range(_NBUF):
        _issue_gather(b, b)

    @pl.loop(0, _NOUTER)
    def _outer(o):
        j0 = o * _NBUF
        for b in range(_NBUF):
            _wait_gather(j0 + b, b)
            _issue_write(j0 + b, b)
        for b in range(_NBUF):
            _wait_write(j0 + b, b)

            @pl.when(o < _NOUTER - 1)
            def _():
                _issue_gather(j0 + b + _NBUF, b)


@functools.partial(
    pl.kernel,
    out_type=jax.ShapeDtypeStruct((_B, _D), jnp.float32),
    mesh=plsc.VectorSubcoreMesh(core_axis_name="c", subcore_axis_name="s"),
    scratch_types=[
        pltpu.VMEM((_NCHUNK, _CB), jnp.int32),
        pltpu.VMEM((_NBUF, _CB, _D), jnp.float32),
        pltpu.SemaphoreType.DMA,
        pltpu.SemaphoreType.DMA,
    ],
)
def _gather_rows(idx_hbm, table_hbm, out_hbm, idx_v, rows_v, gsem, wsem):
    _gather_body(idx_hbm, table_hbm, out_hbm, idx_v, rows_v, gsem, wsem)


def kernel(x, weight):
    scaled = _scale_table(weight)
    idx2d = x.reshape(_B // _CB, _CB)
    out = _gather_rows(idx2d, scaled)
    return out.reshape(4096, 200, _D)


# speedup vs baseline: 3.9392x; 3.9392x over previous
import functools
import math

import jax
import jax.numpy as jnp
from jax import lax
from jax.experimental import pallas as pl
from jax.experimental.pallas import tpu as pltpu
from jax.experimental.pallas import tpu_sc as plsc

_VOCAB = 100000
_D = 64
_SCALE = math.sqrt(_D)

_NC = 2
_NS = 16
_NW = _NC * _NS

_B = 4096 * 200
_CB = 128
_ROWS_PER_W = _B // _NW
_NCHUNK = _ROWS_PER_W // _CB
_NBUF = 4
_NOUTER = _NCHUNK // _NBUF


def _scale_body(w_ref, o_ref):
    o_ref[...] = w_ref[...] * _SCALE


def _scale_table(weight):
    return pl.pallas_call(
        _scale_body,
        out_shape=jax.ShapeDtypeStruct((_VOCAB, _D), jnp.float32),
        grid=(25,),
        in_specs=[pl.BlockSpec((_VOCAB // 25, _D), lambda i: (i, 0))],
        out_specs=pl.BlockSpec((_VOCAB // 25, _D), lambda i: (i, 0)),
    )(weight)


def _gather_body(idx_hbm, table_hbm, out_hbm, idx_v, rows_v, gsem, wsem):
    wid = lax.axis_index("s") * _NC + lax.axis_index("c")
    base_row = wid * _ROWS_PER_W

    pltpu.sync_copy(idx_hbm.at[pl.ds(wid * _NCHUNK, _NCHUNK)], idx_v)

    def _issue_gather(j, b):
        pltpu.async_copy(table_hbm.at[idx_v.at[j]], rows_v.at[b], gsem)

    def _wait_gather(j, b):
        pltpu.make_async_copy(table_hbm.at[idx_v.at[j]], rows_v.at[b], gsem).wait()

    def _issue_write(j, b):
        pltpu.async_copy(rows_v.at[b], out_hbm.at[pl.ds(base_row + j * _CB, _CB)], wsem)

    def _wait_write(j, b):
        pltpu.make_async_copy(rows_v.at[b], out_hbm.at[pl.ds(base_row + j * _CB, _CB)], wsem).wait()

    for b in range(_NBUF):
        _issue_gather(b, b)

    @pl.loop(0, _NOUTER)
    def _outer(o):
        j0 = o * _NBUF
        for b in range(_NBUF):
            _wait_gather(j0 + b, b)
            _issue_write(j0 + b, b)
        for b in range(_NBUF):
            _wait_write(j0 + b, b)

            @pl.when(o < _NOUTER - 1)
            def _():
                _issue_gather(j0 + b + _NBUF, b)


@functools.partial(
    pl.kernel,
    out_type=jax.ShapeDtypeStruct((_B, _D), jnp.float32),
    mesh=plsc.VectorSubcoreMesh(core_axis_name="c", subcore_axis_name="s"),
    compiler_params=pltpu.CompilerParams(use_tc_tiling_on_sc=False),
    scratch_types=[
        pltpu.VMEM((_NCHUNK, _CB), jnp.int32),
        pltpu.VMEM((_NBUF, _CB, _D), jnp.float32),
        pltpu.SemaphoreType.DMA,
        pltpu.SemaphoreType.DMA,
    ],
)
def _gather_rows(idx_hbm, table_hbm, out_hbm, idx_v, rows_v, gsem, wsem):
    _gather_body(idx_hbm, table_hbm, out_hbm, idx_v, rows_v, gsem, wsem)


def kernel(x, weight):
    scaled = _scale_table(weight)
    idx2d = x.reshape(_B // _CB, _CB)
    out = _gather_rows(idx2d, scaled)
    return out.reshape(4096, 200, _D)
